# baseline (speedup 1.0000x reference)
import functools

import jax
import jax.numpy as jnp
from jax import lax
from jax.experimental import pallas as pl
from jax.experimental.pallas import tpu as pltpu
from jax.experimental.pallas import tpu_sc as plsc

NC = 2
NS = 16
L = 16
W = 128


def _gather_body(tab_hbm, vidx_hbm, out_hbm, vidx_v, idx_a, idx_b, off_v,
                 buf_a, buf_b, outw, sem, *, V, K, R, RH, N8):
    wid = lax.axis_index("s") * NC + lax.axis_index("c")
    base = wid * R
    nsl = RH // L
    nch = RH // W
    per_row = W // L

    pltpu.sync_copy(vidx_hbm, vidx_v)

    for h in range(R // RH):
        hbase = base + h * RH

        def idx_body(t, carry):
            g = hbase + t * L + lax.iota(jnp.int32, L)
            b = lax.div(g, jnp.int32(K))
            k = g - b * K
            vk = plsc.load_gather(vidx_v, [k])
            w = (b * V + vk) * 3
            ca = lax.shift_right_logical(w, 3)
            idx_a[t // per_row, pl.ds((t % per_row) * L, L)] = ca
            idx_b[t // per_row, pl.ds((t % per_row) * L, L)] = (
                jnp.minimum(ca + 1, N8 - 1))
            off_v[pl.ds(t * L, L)] = lax.bitwise_and(w, 7)
            return carry

        lax.fori_loop(0, nsl, idx_body, 0)

        def start_a(c, carry):
            pltpu.make_async_copy(
                tab_hbm.at[idx_a.at[c]], buf_a.at[pl.ds(c * W, W)], sem
            ).start()
            return carry

        def wait_a(c, carry):
            pltpu.make_async_copy(
                tab_hbm.at[idx_a.at[c]], buf_a.at[pl.ds(c * W, W)], sem
            ).wait()
            return carry

        def start_b(c, carry):
            pltpu.make_async_copy(
                tab_hbm.at[idx_b.at[c]], buf_b.at[pl.ds(c * W, W)], sem
            ).start()
            return carry

        def wait_b(c, carry):
            pltpu.make_async_copy(
                tab_hbm.at[idx_b.at[c]], buf_b.at[pl.ds(c * W, W)], sem
            ).wait()
            return carry

        lax.fori_loop(0, nch, start_a, 0)
        lax.fori_loop(0, nch, wait_a, 0)
        lax.fori_loop(0, nch, start_b, 0)
        lax.fori_loop(0, nch, wait_b, 0)

        def ext_body(t, carry):
            rvec = t * L + lax.iota(jnp.int32, L)
            offs = off_v[pl.ds(t * L, L)]
            for j in range(3):
                s = offs + j
                sa = jnp.minimum(s, 7)
                sb = jnp.maximum(s - 8, 0)
                va = plsc.load_gather(buf_a, [rvec, sa])
                vb = plsc.load_gather(buf_b, [rvec, sb])
                val = jnp.where(s < 8, va, vb)
                plsc.store_scatter(outw, [rvec * 3 + j], val)
            return carry

        lax.fori_loop(0, nsl, ext_body, 0)

        pltpu.sync_copy(outw, out_hbm.at[pl.ds(hbase * 3, RH * 3)])


def kernel(vertices, vert_idx):
    B, V, C = vertices.shape
    (K,) = vert_idx.shape
    NW = NC * NS
    R = (B * K) // NW
    RH = R // 2
    NWORDS = B * V * C
    N8 = NWORDS // 8
    assert (B * K) % NW == 0 and RH % W == 0 and NWORDS % 8 == 0

    tab = vertices.reshape(N8, 8)
    vidx = vert_idx.astype(jnp.int32)

    mesh = plsc.VectorSubcoreMesh(
        core_axis_name="c", subcore_axis_name="s",
        num_cores=NC, num_subcores=NS,
    )
    flat = pl.kernel(
        functools.partial(_gather_body, V=V, K=K, R=R, RH=RH, N8=N8),
        out_type=jax.ShapeDtypeStruct((B * K * C,), jnp.float32),
        mesh=mesh,
        scratch_types=[
            pltpu.VMEM((K,), jnp.int32),
            pltpu.VMEM((RH // W, W), jnp.int32),
            pltpu.VMEM((RH // W, W), jnp.int32),
            pltpu.VMEM((RH,), jnp.int32),
            pltpu.VMEM((RH, 8), jnp.float32),
            pltpu.VMEM((RH, 8), jnp.float32),
            pltpu.VMEM((RH * C,), jnp.float32),
            pltpu.SemaphoreType.DMA,
        ],
        compiler_params=pltpu.CompilerParams(
            use_tc_tiling_on_sc=False, needs_layout_passes=False,
        ),
    )(tab, vidx)
    return flat.reshape(B, K, C)


# speedup vs baseline: 1.0106x; 1.0106x over previous
import functools

import jax
import jax.numpy as jnp
from jax import lax
from jax.experimental import pallas as pl
from jax.experimental.pallas import tpu as pltpu
from jax.experimental.pallas import tpu_sc as plsc

NC = 2
NS = 16
L = 16
W = 128
G = 23


def _gather_body(tab_hbm, vidx_hbm, out_hbm, vidx_v, idx_v, out_v, sem,
                 *, V, K, OW):
    wid = lax.axis_index("s") * NC + lax.axis_index("c")
    base = wid * OW
    nsl = OW // L
    nch = OW // W
    per_row = W // L

    pltpu.sync_copy(vidx_hbm, vidx_v)

    def idx_body(t, carry):
        o = base + t * L + lax.iota(jnp.int32, L)
        r = lax.div(o, jnp.int32(3))
        j = o - r * 3
        b = lax.div(r, jnp.int32(K))
        k = r - b * K
        vk = plsc.load_gather(vidx_v, [k])
        w = (b * V + vk) * 3 + j
        idx_v[t // per_row, pl.ds((t % per_row) * L, L)] = w
        return carry

    lax.fori_loop(0, nsl, idx_body, 0)

    def start_body(c, carry):
        pltpu.make_async_copy(
            tab_hbm.at[idx_v.at[c]], out_v.at[pl.ds(c * W, W)], sem
        ).start()
        return carry

    def wait_body(c, carry):
        pltpu.make_async_copy(
            tab_hbm.at[idx_v.at[c]], out_v.at[pl.ds(c * W, W)], sem
        ).wait()
        return carry

    for g in range(nch // G):
        lax.fori_loop(g * G, (g + 1) * G, start_body, 0)
        lax.fori_loop(g * G, (g + 1) * G, wait_body, 0)

    pltpu.sync_copy(out_v, out_hbm.at[pl.ds(base, OW)])


def kernel(vertices, vert_idx):
    B, V, C = vertices.shape
    (K,) = vert_idx.shape
    NW = NC * NS
    OW = (B * K * C) // NW
    assert (B * K * C) % NW == 0 and OW % W == 0 and (OW // W) % G == 0

    tab = vertices.reshape(B * V * C)
    vidx = vert_idx.astype(jnp.int32)

    mesh = plsc.VectorSubcoreMesh(
        core_axis_name="c", subcore_axis_name="s",
        num_cores=NC, num_subcores=NS,
    )
    flat = pl.kernel(
        functools.partial(_gather_body, V=V, K=K, OW=OW),
        out_type=jax.ShapeDtypeStruct((B * K * C,), jnp.float32),
        mesh=mesh,
        scratch_types=[
            pltpu.VMEM((K,), jnp.int32),
            pltpu.VMEM((OW // W, W), jnp.int32),
            pltpu.VMEM((OW,), jnp.float32),
            pltpu.SemaphoreType.DMA,
        ],
        compiler_params=pltpu.CompilerParams(
            use_tc_tiling_on_sc=False, needs_layout_passes=False,
        ),
    )(tab, vidx)
    return flat.reshape(B, K, C)


# speedup vs baseline: 4747.9220x; 4697.9000x over previous
import functools

import jax
import jax.numpy as jnp
from jax import lax
from jax.experimental import pallas as pl
from jax.experimental.pallas import tpu as pltpu

GJ = 8


def _gather_block_body(idx_s, *refs):
    ins = refs[:-1]
    out_ref = refs[-1]
    g = pl.program_id(0)
    for j, in_ref in enumerate(ins):
        m = idx_s[GJ * g + j] % GJ
        onehot = (lax.iota(jnp.int32, GJ) == m).astype(jnp.float32)
        out_ref[:, j, :] = jnp.sum(in_ref[...] * onehot[None, :, None], axis=1)


def _band_spec(j, B, C):
    return pl.BlockSpec(
        (C, GJ, B),
        functools.partial(lambda jj, g, idx: (0, idx[GJ * g + jj] // GJ, 0), j),
    )


def kernel(vertices, vert_idx):
    B, V, C = vertices.shape
    (K,) = vert_idx.shape
    KP = ((K + GJ - 1) // GJ) * GJ
    idxp = jnp.concatenate(
        [vert_idx.astype(jnp.int32), jnp.zeros((KP - K,), jnp.int32)])
    vt = jnp.transpose(vertices, (2, 1, 0))
    out_t = pl.pallas_call(
        _gather_block_body,
        grid_spec=pltpu.PrefetchScalarGridSpec(
            num_scalar_prefetch=1,
            grid=(KP // GJ,),
            in_specs=[_band_spec(j, B, C) for j in range(GJ)],
            out_specs=pl.BlockSpec((C, GJ, B), lambda g, idx: (0, g, 0)),
        ),
        out_shape=jax.ShapeDtypeStruct((C, K, B), jnp.float32),
    )(idxp, *([vt] * GJ))
    return jnp.transpose(out_t, (2, 1, 0))
